# baseline (speedup 1.0000x reference)
import functools

import jax
import jax.numpy as jnp
from jax import lax
from jax.experimental import pallas as pl
from jax.experimental.pallas import tpu as pltpu
from jax.experimental.pallas import tpu_sc as plsc

N = 10000
E = 160000
D = 256
QW = 64
NCLS = 40
NCP = 64

NC = 2
NS = 16

NPAD = 10240
RPT = NPAD // NS
CHUNK = 128
NCHUNKS = E // CHUNK

_mesh = plsc.VectorSubcoreMesh(core_axis_name="c", subcore_axis_name="s")
_SCPARAMS = pltpu.CompilerParams(use_tc_tiling_on_sc=False)

f32 = jnp.float32
i32 = jnp.int32


def _chunk_span(s, base, q, r):
  cb = base + s * q + jnp.minimum(s, r)
  cnt = jnp.where(s < r, q + 1, q)
  return cb, cnt


def _preload(src2d, buf, s, cb, q, r):
  @pl.when(s < r)
  def _():
    pltpu.sync_copy(src2d.at[pl.ds(cb, q + 1)], buf.at[pl.ds(0, q + 1)])

  @pl.when(s >= r)
  def _():
    pltpu.sync_copy(src2d.at[pl.ds(cb, q)], buf.at[pl.ds(0, q)])


def _gather_scatter(table, acc, sidx_all, didx_all, rows, gsems, ssems,
                    cnt):
  depth = len(rows)
  ngroups = cnt // depth
  rem = cnt - ngroups * depth

  def body(j, _):
    k = depth * j
    gds = [pltpu.async_copy(table.at[sidx_all.at[k + i]], rows[i], gsems[i])
           for i in range(depth)]
    sds = []
    for i in range(depth):
      gds[i].wait()
      sds.append(pltpu.async_copy(rows[i], acc.at[didx_all.at[k + i]],
                                  ssems[i], add=True))
    for i in range(depth):
      sds[i].wait()
    return 0

  lax.fori_loop(0, ngroups, body, 0)

  kbase = ngroups * depth
  for i in range(depth - 1):
    @pl.when(rem >= i + 1)
    def _(i=i):
      pltpu.async_copy(table.at[sidx_all.at[kbase + i]],
                       rows[i], gsems[i]).wait()
      pltpu.sync_copy(rows[i], acc.at[didx_all.at[kbase + i]], add=True)


HRPT = RPT // 2


def _init_acc(zeros_hbm, zbuf, acc, row0):
  pltpu.sync_copy(zeros_hbm, zbuf)
  pltpu.sync_copy(zbuf, acc.at[pl.ds(row0, HRPT)])
  pltpu.sync_copy(zbuf, acc.at[pl.ds(row0 + HRPT, HRPT)])


def _writeback(acc, zbuf, out_ref, row0):
  pltpu.sync_copy(acc.at[pl.ds(row0, HRPT)], zbuf)
  pltpu.sync_copy(zbuf, out_ref.at[pl.ds(row0, HRPT)])
  pltpu.sync_copy(acc.at[pl.ds(row0 + HRPT, HRPT)], zbuf)
  pltpu.sync_copy(zbuf, out_ref.at[pl.ds(row0 + HRPT, HRPT)])


DEGW = 16
_QD = (NCHUNKS // NC) // NS
_RD = (NCHUNKS // NC) % NS


@functools.partial(
    pl.kernel,
    out_type=[
        jax.ShapeDtypeStruct((NPAD, DEGW), f32),
        jax.ShapeDtypeStruct((NPAD, DEGW), f32),
    ],
    mesh=_mesh,
    compiler_params=_SCPARAMS,
    scratch_types=[
        pltpu.VMEM_SHARED((NPAD, DEGW), f32),
        pltpu.VMEM((RPT // 2, DEGW), f32),
        pltpu.VMEM((_QD + 1, CHUNK), i32),
        pltpu.VMEM((CHUNK, DEGW), f32),
    ],
)
def _deg_kernel(dst2d, zeros16, ones_hbm, out0, out1, acc, zbuf, didx_all,
                ones):
  c = lax.axis_index("c")
  s = lax.axis_index("s")

  row0 = s * RPT
  _init_acc(zeros16, zbuf, acc, row0)
  pltpu.sync_copy(ones_hbm, ones)
  plsc.subcore_barrier()

  def run(cid, out_ref):
    cb, cnt = _chunk_span(s, cid * (NCHUNKS // NC), _QD, _RD)
    _preload(dst2d, didx_all, s, cb, _QD, _RD)

    def body(j, _):
      pltpu.sync_copy(ones, acc.at[didx_all.at[j]], add=True)
      return 0

    lax.fori_loop(0, cnt, body, 0)

    plsc.subcore_barrier()
    _writeback(acc, zbuf, out_ref, row0)

  @pl.when(c == 0)
  def _():
    run(0, out0)

  @pl.when(c == 1)
  def _():
    run(1, out1)


_QF = NCHUNKS // NS
_RF = NCHUNKS % NS


@functools.partial(
    pl.kernel,
    out_type=[
        jax.ShapeDtypeStruct((NPAD, QW), f32),
        jax.ShapeDtypeStruct((NPAD, QW), f32),
    ],
    mesh=_mesh,
    compiler_params=_SCPARAMS,
    scratch_types=[
        pltpu.VMEM_SHARED((NPAD, QW), f32),
        pltpu.VMEM((RPT // 2, QW), f32),
        pltpu.VMEM((_QF + 1, CHUNK), i32),
        pltpu.VMEM((_QF + 1, CHUNK), i32),
        pltpu.VMEM((CHUNK, QW), f32),
        pltpu.VMEM((CHUNK, QW), f32),
        pltpu.VMEM((CHUNK, QW), f32),
        pltpu.VMEM((CHUNK, QW), f32),
        pltpu.VMEM((CHUNK, QW), f32),
        pltpu.VMEM((CHUNK, QW), f32),
        [pltpu.SemaphoreType.DMA] * 6,
        [pltpu.SemaphoreType.DMA] * 6,
    ],
)
def _aggq_kernel(src2d, dst2d, qa, qb, zeros64, out_qa, out_qb,
                 acc, zbuf, sidx_all, didx_all,
                 rows0, rows1, rows2, rows3, rows4, rows5, gsems, ssems):
  c = lax.axis_index("c")
  s = lax.axis_index("s")

  row0 = s * RPT
  _init_acc(zeros64, zbuf, acc, row0)
  plsc.subcore_barrier()

  cb, cnt = _chunk_span(s, 0, _QF, _RF)
  _preload(src2d, sidx_all, s, cb, _QF, _RF)
  _preload(dst2d, didx_all, s, cb, _QF, _RF)

  rows = [rows0, rows1, rows2, rows3, rows4, rows5]

  def run(h_ref, out_ref):
    _gather_scatter(h_ref, acc, sidx_all, didx_all, rows,
                    gsems, ssems, cnt)
    plsc.subcore_barrier()
    _writeback(acc, zbuf, out_ref, row0)

  @pl.when(c == 0)
  def _():
    run(qa, out_qa)

  @pl.when(c == 1)
  def _():
    run(qb, out_qb)


@functools.partial(
    pl.kernel,
    out_type=[
        jax.ShapeDtypeStruct((NPAD, NCP), f32),
        jax.ShapeDtypeStruct((NPAD, NCP), f32),
    ],
    mesh=_mesh,
    compiler_params=_SCPARAMS,
    scratch_types=[
        pltpu.VMEM_SHARED((NPAD, NCP), f32),
        pltpu.VMEM((RPT // 2, NCP), f32),
        pltpu.VMEM((_QD + 1, CHUNK), i32),
        pltpu.VMEM((_QD + 1, CHUNK), i32),
        pltpu.VMEM((CHUNK, NCP), f32),
        pltpu.VMEM((CHUNK, NCP), f32),
        pltpu.VMEM((CHUNK, NCP), f32),
        pltpu.VMEM((CHUNK, NCP), f32),
        pltpu.VMEM((CHUNK, NCP), f32),
        pltpu.VMEM((CHUNK, NCP), f32),
        [pltpu.SemaphoreType.DMA] * 6,
        [pltpu.SemaphoreType.DMA] * 6,
    ],
)
def _agg64_kernel(src2d, dst2d, g, zeros64, out_p0, out_p1,
                  acc, zbuf, sidx_all, didx_all,
                  rows0, rows1, rows2, rows3, rows4, rows5, gsems, ssems):
  c = lax.axis_index("c")
  s = lax.axis_index("s")

  row0 = s * RPT
  _init_acc(zeros64, zbuf, acc, row0)
  plsc.subcore_barrier()

  rows = [rows0, rows1, rows2, rows3, rows4, rows5]

  def run(cid, out_ref):
    cb, cnt = _chunk_span(s, cid * (NCHUNKS // NC), _QD, _RD)
    _preload(src2d, sidx_all, s, cb, _QD, _RD)
    _preload(dst2d, didx_all, s, cb, _QD, _RD)

    _gather_scatter(g, acc, sidx_all, didx_all, rows, gsems, ssems, cnt)
    plsc.subcore_barrier()
    _writeback(acc, zbuf, out_ref, row0)

  @pl.when(c == 0)
  def _():
    run(0, out_p0)

  @pl.when(c == 1)
  def _():
    run(1, out_p1)


BLK = 1000
GRID = N // BLK


def _norm_scale_body(dega, degb, x, norm_o, q0, q1, q2, q3):
  d = dega[...][:, :1] + degb[...][:, :1]
  nrm = jnp.where(d > 0, lax.rsqrt(jnp.maximum(d, 1.0)), 0.0)
  norm_o[...] = nrm
  hs = x[...] * nrm
  q0[...] = hs[:, 0 * QW:1 * QW]
  q1[...] = hs[:, 1 * QW:2 * QW]
  q2[...] = hs[:, 2 * QW:3 * QW]
  q3[...] = hs[:, 3 * QW:4 * QW]


def _norm_scale(dega, degb, x):
  return pl.pallas_call(
      _norm_scale_body,
      grid=(GRID,),
      in_specs=[
          pl.BlockSpec((BLK, DEGW), lambda i: (i, 0)),
          pl.BlockSpec((BLK, DEGW), lambda i: (i, 0)),
          pl.BlockSpec((BLK, D), lambda i: (i, 0)),
      ],
      out_specs=[
          pl.BlockSpec((BLK, 1), lambda i: (i, 0)),
          pl.BlockSpec((BLK, QW), lambda i: (i, 0)),
          pl.BlockSpec((BLK, QW), lambda i: (i, 0)),
          pl.BlockSpec((BLK, QW), lambda i: (i, 0)),
          pl.BlockSpec((BLK, QW), lambda i: (i, 0)),
      ],
      out_shape=[
          jax.ShapeDtypeStruct((N, 1), f32),
          jax.ShapeDtypeStruct((N, QW), f32),
          jax.ShapeDtypeStruct((N, QW), f32),
          jax.ShapeDtypeStruct((N, QW), f32),
          jax.ShapeDtypeStruct((N, QW), f32),
      ],
  )(dega, degb, x)


_QSPEC = pl.BlockSpec((BLK, QW), lambda i: (i, 0))
_NSPEC = pl.BlockSpec((BLK, 1), lambda i: (i, 0))


def _partial_body(a0, a1, norm, w, z_o):
  h = jnp.concatenate([a0[...], a1[...]], axis=1) * norm[...]
  z_o[...] = jnp.dot(h, w[...], preferred_element_type=f32)


def _partial(a0, a1, norm, w):
  return pl.pallas_call(
      _partial_body,
      grid=(GRID,),
      in_specs=[
          _QSPEC, _QSPEC, _NSPEC,
          pl.BlockSpec((D // 2, D), lambda i: (0, 0)),
      ],
      out_specs=pl.BlockSpec((BLK, D), lambda i: (i, 0)),
      out_shape=jax.ShapeDtypeStruct((N, D), f32),
  )(a0, a1, norm, w)


def _finish0_body(z, a2, a3, norm, w, b, q0, q1, q2, q3):
  h = jnp.concatenate([a2[...], a3[...]], axis=1) * norm[...]
  u = z[...] + jnp.dot(h, w[...], preferred_element_type=f32) + b[...]
  h1 = jnp.maximum(u, 0.0) * norm[...]
  q0[...] = h1[:, 0 * QW:1 * QW]
  q1[...] = h1[:, 1 * QW:2 * QW]
  q2[...] = h1[:, 2 * QW:3 * QW]
  q3[...] = h1[:, 3 * QW:4 * QW]


def _finish0(z, a2, a3, norm, w, b):
  return pl.pallas_call(
      _finish0_body,
      grid=(GRID,),
      in_specs=[
          pl.BlockSpec((BLK, D), lambda i: (i, 0)),
          _QSPEC, _QSPEC, _NSPEC,
          pl.BlockSpec((D // 2, D), lambda i: (1, 0)),
          pl.BlockSpec((1, D), lambda i: (0, 0)),
      ],
      out_specs=[_QSPEC, _QSPEC, _QSPEC, _QSPEC],
      out_shape=[jax.ShapeDtypeStruct((N, QW), f32)] * 4,
  )(z, a2, a3, norm, w, b)


def _finish1_body(z, a2, a3, norm, w1, b1, w2, g_o):
  h = jnp.concatenate([a2[...], a3[...]], axis=1) * norm[...]
  u = z[...] + jnp.dot(h, w1[...], preferred_element_type=f32) + b1[...]
  h2 = jnp.maximum(u, 0.0)
  g = jnp.dot(h2, w2[...], preferred_element_type=f32)
  g_o[...] = g * norm[...]


def _finish1(z, a2, a3, norm, w1, b1, w2):
  return pl.pallas_call(
      _finish1_body,
      grid=(GRID,),
      in_specs=[
          pl.BlockSpec((BLK, D), lambda i: (i, 0)),
          _QSPEC, _QSPEC, _NSPEC,
          pl.BlockSpec((D // 2, D), lambda i: (1, 0)),
          pl.BlockSpec((1, D), lambda i: (0, 0)),
          pl.BlockSpec((D, NCP), lambda i: (0, 0)),
      ],
      out_specs=pl.BlockSpec((BLK, NCP), lambda i: (i, 0)),
      out_shape=jax.ShapeDtypeStruct((N, NCP), f32),
  )(z, a2, a3, norm, w1, b1, w2)


def _final_body(a0, a1, norm, b2, out_o):
  out_o[...] = (a0[...] + a1[...]) * norm[...] + b2[...]


def _final(a0, a1, norm, b2):
  return pl.pallas_call(
      _final_body,
      grid=(GRID,),
      in_specs=[
          pl.BlockSpec((BLK, NCP), lambda i: (i, 0)),
          pl.BlockSpec((BLK, NCP), lambda i: (i, 0)),
          _NSPEC,
          pl.BlockSpec((1, NCP), lambda i: (0, 0)),
      ],
      out_specs=pl.BlockSpec((BLK, NCP), lambda i: (i, 0)),
      out_shape=jax.ShapeDtypeStruct((N, NCP), f32),
  )(a0, a1, norm, b2)


def kernel(x, edge_index, W0, b0, W1, b1, W2, b2):
  src2d = edge_index[0].reshape(NCHUNKS, CHUNK)
  dst2d = edge_index[1].reshape(NCHUNKS, CHUNK)

  W2p = jnp.pad(W2, ((0, 0), (0, NCP - NCLS)))
  b2p = jnp.pad(b2, (0, NCP - NCLS)).reshape(1, NCP)

  zeros16 = jnp.zeros((RPT // 2, DEGW), f32)
  zeros64 = jnp.zeros((RPT // 2, QW), f32)
  ones_hbm = jnp.ones((CHUNK, DEGW), f32)

  deg_p0, deg_p1 = _deg_kernel(dst2d, zeros16, ones_hbm)
  norm, h0, h1, h2, h3 = _norm_scale(deg_p0, deg_p1, x)

  a0, a1 = _aggq_kernel(src2d, dst2d, h0, h1, zeros64)
  a2, a3 = _aggq_kernel(src2d, dst2d, h2, h3, zeros64)
  z0 = _partial(a0, a1, norm, W0)
  u0, u1, u2, u3 = _finish0(z0, a2, a3, norm, W0, b0.reshape(1, D))

  b0_, b1_ = _aggq_kernel(src2d, dst2d, u0, u1, zeros64)
  b2_, b3_ = _aggq_kernel(src2d, dst2d, u2, u3, zeros64)
  z1 = _partial(b0_, b1_, norm, W1)
  g = _finish1(z1, b2_, b3_, norm, W1, b1.reshape(1, D), W2p)

  p0, p1 = _agg64_kernel(src2d, dst2d, g, zeros64)
  out = _final(p0, p1, norm, b2p)
  return out[:, :NCLS]


# speedup vs baseline: 1.1582x; 1.1582x over previous
import functools

import jax
import jax.numpy as jnp
from jax import lax
from jax.experimental import pallas as pl
from jax.experimental.pallas import tpu as pltpu
from jax.experimental.pallas import tpu_sc as plsc

N = 10000
E = 160000
D = 256
QW = 64
NCLS = 40
NCP = 64

NC = 2
NS = 16

NPAD = 10240
RPT = NPAD // NS
CHUNK = 128
NCHUNKS = E // CHUNK

_mesh = plsc.VectorSubcoreMesh(core_axis_name="c", subcore_axis_name="s")
_SCPARAMS = pltpu.CompilerParams(use_tc_tiling_on_sc=False)

f32 = jnp.float32
i32 = jnp.int32


def _chunk_span(s, base, q, r):
  cb = base + s * q + jnp.minimum(s, r)
  cnt = jnp.where(s < r, q + 1, q)
  return cb, cnt


def _preload(src2d, buf, s, cb, q, r):
  @pl.when(s < r)
  def _():
    pltpu.sync_copy(src2d.at[pl.ds(cb, q + 1)], buf.at[pl.ds(0, q + 1)])

  @pl.when(s >= r)
  def _():
    pltpu.sync_copy(src2d.at[pl.ds(cb, q)], buf.at[pl.ds(0, q)])


def _gather_scatter(table, acc, sidx_all, didx_all, rows, gsems, ssems,
                    cnt):
  depth = len(rows)
  ngroups = cnt // depth
  rem = cnt - ngroups * depth

  def body(j, _):
    k = depth * j
    gds = [pltpu.async_copy(table.at[sidx_all.at[k + i]], rows[i], gsems[i])
           for i in range(depth)]
    sds = []
    for i in range(depth):
      gds[i].wait()
      sds.append(pltpu.async_copy(rows[i], acc.at[didx_all.at[k + i]],
                                  ssems[i], add=True))
    for i in range(depth):
      sds[i].wait()
    return 0

  lax.fori_loop(0, ngroups, body, 0)

  kbase = ngroups * depth
  for i in range(depth - 1):
    @pl.when(rem >= i + 1)
    def _(i=i):
      pltpu.async_copy(table.at[sidx_all.at[kbase + i]],
                       rows[i], gsems[i]).wait()
      pltpu.sync_copy(rows[i], acc.at[didx_all.at[kbase + i]], add=True)


HRPT = RPT // 2


def _init_acc(zeros_hbm, zbuf, acc, row0):
  pltpu.sync_copy(zeros_hbm, zbuf)
  pltpu.sync_copy(zbuf, acc.at[pl.ds(row0, HRPT)])
  pltpu.sync_copy(zbuf, acc.at[pl.ds(row0 + HRPT, HRPT)])


def _writeback(acc, zbuf, out_ref, row0, col0, ncols):
  pltpu.sync_copy(acc.at[pl.ds(row0, HRPT)], zbuf)
  pltpu.sync_copy(zbuf, out_ref.at[pl.ds(row0, HRPT), pl.ds(col0, ncols)])
  pltpu.sync_copy(acc.at[pl.ds(row0 + HRPT, HRPT)], zbuf)
  pltpu.sync_copy(
      zbuf, out_ref.at[pl.ds(row0 + HRPT, HRPT), pl.ds(col0, ncols)])


DEGW = 16
_QD = (NCHUNKS // NC) // NS
_RD = (NCHUNKS // NC) % NS


@functools.partial(
    pl.kernel,
    out_type=jax.ShapeDtypeStruct((NPAD, 2 * DEGW), f32),
    mesh=_mesh,
    compiler_params=_SCPARAMS,
    scratch_types=[
        pltpu.VMEM_SHARED((NPAD, DEGW), f32),
        pltpu.VMEM((RPT // 2, DEGW), f32),
        pltpu.VMEM((_QD + 1, CHUNK), i32),
        pltpu.VMEM((CHUNK, DEGW), f32),
    ],
)
def _deg_kernel(dst2d, zeros16, ones_hbm, out, acc, zbuf, didx_all,
                ones):
  c = lax.axis_index("c")
  s = lax.axis_index("s")

  row0 = s * RPT
  _init_acc(zeros16, zbuf, acc, row0)
  pltpu.sync_copy(ones_hbm, ones)
  plsc.subcore_barrier()

  def run(cid):
    cb, cnt = _chunk_span(s, cid * (NCHUNKS // NC), _QD, _RD)
    _preload(dst2d, didx_all, s, cb, _QD, _RD)

    def body(j, _):
      pltpu.sync_copy(ones, acc.at[didx_all.at[j]], add=True)
      return 0

    lax.fori_loop(0, cnt, body, 0)

    plsc.subcore_barrier()
    _writeback(acc, zbuf, out, row0, cid * DEGW, DEGW)

  @pl.when(c == 0)
  def _():
    run(0)

  @pl.when(c == 1)
  def _():
    run(1)


_QF = NCHUNKS // NS
_RF = NCHUNKS % NS


@functools.partial(
    pl.kernel,
    out_type=jax.ShapeDtypeStruct((NPAD, 2 * QW), f32),
    mesh=_mesh,
    compiler_params=_SCPARAMS,
    scratch_types=[
        pltpu.VMEM_SHARED((NPAD, QW), f32),
        pltpu.VMEM((RPT // 2, QW), f32),
        pltpu.VMEM((_QF + 1, CHUNK), i32),
        pltpu.VMEM((_QF + 1, CHUNK), i32),
        pltpu.VMEM((CHUNK, QW), f32),
        pltpu.VMEM((CHUNK, QW), f32),
        pltpu.VMEM((CHUNK, QW), f32),
        pltpu.VMEM((CHUNK, QW), f32),
        pltpu.VMEM((CHUNK, QW), f32),
        pltpu.VMEM((CHUNK, QW), f32),
        [pltpu.SemaphoreType.DMA] * 6,
        [pltpu.SemaphoreType.DMA] * 6,
    ],
)
def _aggq_kernel(src2d, dst2d, qa, qb, zeros64, out,
                 acc, zbuf, sidx_all, didx_all,
                 rows0, rows1, rows2, rows3, rows4, rows5, gsems, ssems):
  c = lax.axis_index("c")
  s = lax.axis_index("s")

  row0 = s * RPT
  _init_acc(zeros64, zbuf, acc, row0)
  plsc.subcore_barrier()

  cb, cnt = _chunk_span(s, 0, _QF, _RF)
  _preload(src2d, sidx_all, s, cb, _QF, _RF)
  _preload(dst2d, didx_all, s, cb, _QF, _RF)

  rows = [rows0, rows1, rows2, rows3, rows4, rows5]

  def run(h_ref, col0):
    _gather_scatter(h_ref, acc, sidx_all, didx_all, rows,
                    gsems, ssems, cnt)
    plsc.subcore_barrier()
    _writeback(acc, zbuf, out, row0, col0, QW)

  @pl.when(c == 0)
  def _():
    run(qa, 0)

  @pl.when(c == 1)
  def _():
    run(qb, QW)


@functools.partial(
    pl.kernel,
    out_type=jax.ShapeDtypeStruct((NPAD, 2 * NCP), f32),
    mesh=_mesh,
    compiler_params=_SCPARAMS,
    scratch_types=[
        pltpu.VMEM_SHARED((NPAD, NCP), f32),
        pltpu.VMEM((RPT // 2, NCP), f32),
        pltpu.VMEM((_QD + 1, CHUNK), i32),
        pltpu.VMEM((_QD + 1, CHUNK), i32),
        pltpu.VMEM((CHUNK, NCP), f32),
        pltpu.VMEM((CHUNK, NCP), f32),
        pltpu.VMEM((CHUNK, NCP), f32),
        pltpu.VMEM((CHUNK, NCP), f32),
        pltpu.VMEM((CHUNK, NCP), f32),
        pltpu.VMEM((CHUNK, NCP), f32),
        [pltpu.SemaphoreType.DMA] * 6,
        [pltpu.SemaphoreType.DMA] * 6,
    ],
)
def _agg64_kernel(src2d, dst2d, g, zeros64, out,
                  acc, zbuf, sidx_all, didx_all,
                  rows0, rows1, rows2, rows3, rows4, rows5, gsems, ssems):
  c = lax.axis_index("c")
  s = lax.axis_index("s")

  row0 = s * RPT
  _init_acc(zeros64, zbuf, acc, row0)
  plsc.subcore_barrier()

  rows = [rows0, rows1, rows2, rows3, rows4, rows5]

  def run(cid):
    cb, cnt = _chunk_span(s, cid * (NCHUNKS // NC), _QD, _RD)
    _preload(src2d, sidx_all, s, cb, _QD, _RD)
    _preload(dst2d, didx_all, s, cb, _QD, _RD)

    _gather_scatter(g, acc, sidx_all, didx_all, rows, gsems, ssems, cnt)
    plsc.subcore_barrier()
    _writeback(acc, zbuf, out, row0, cid * NCP, NCP)

  @pl.when(c == 0)
  def _():
    run(0)

  @pl.when(c == 1)
  def _():
    run(1)


BLK = 2000
GRID = N // BLK


def _norm_scale_body(deg, x, norm_o, q0, q1, q2, q3):
  d = deg[...][:, :1] + deg[...][:, DEGW:DEGW + 1]
  nrm = jnp.where(d > 0, lax.rsqrt(jnp.maximum(d, 1.0)), 0.0)
  norm_o[...] = nrm
  hs = x[...] * nrm
  q0[...] = hs[:, 0 * QW:1 * QW]
  q1[...] = hs[:, 1 * QW:2 * QW]
  q2[...] = hs[:, 2 * QW:3 * QW]
  q3[...] = hs[:, 3 * QW:4 * QW]


def _norm_scale(deg, x):
  return pl.pallas_call(
      _norm_scale_body,
      grid=(GRID,),
      in_specs=[
          pl.BlockSpec((BLK, 2 * DEGW), lambda i: (i, 0)),
          pl.BlockSpec((BLK, D), lambda i: (i, 0)),
      ],
      out_specs=[
          pl.BlockSpec((BLK, 1), lambda i: (i, 0)),
          pl.BlockSpec((BLK, QW), lambda i: (i, 0)),
          pl.BlockSpec((BLK, QW), lambda i: (i, 0)),
          pl.BlockSpec((BLK, QW), lambda i: (i, 0)),
          pl.BlockSpec((BLK, QW), lambda i: (i, 0)),
      ],
      out_shape=[
          jax.ShapeDtypeStruct((N, 1), f32),
          jax.ShapeDtypeStruct((N, QW), f32),
          jax.ShapeDtypeStruct((N, QW), f32),
          jax.ShapeDtypeStruct((N, QW), f32),
          jax.ShapeDtypeStruct((N, QW), f32),
      ],
  )(deg, x)


_QSPEC = pl.BlockSpec((BLK, QW), lambda i: (i, 0))
_NSPEC = pl.BlockSpec((BLK, 1), lambda i: (i, 0))


def _layer0_body(a01, a23, norm, w, b, q0, q1, q2, q3):
  h = jnp.concatenate([a01[...], a23[...]], axis=1)
  h = h * norm[...]
  z = jnp.dot(h, w[...],
              preferred_element_type=f32) + b[...]
  h1 = jnp.maximum(z, 0.0) * norm[...]
  q0[...] = h1[:, 0 * QW:1 * QW]
  q1[...] = h1[:, 1 * QW:2 * QW]
  q2[...] = h1[:, 2 * QW:3 * QW]
  q3[...] = h1[:, 3 * QW:4 * QW]


_HSPEC = pl.BlockSpec((BLK, 2 * QW), lambda i: (i, 0))


def _layer0(a01, a23, norm, w, b):
  return pl.pallas_call(
      _layer0_body,
      grid=(GRID,),
      in_specs=[
          _HSPEC, _HSPEC, _NSPEC,
          pl.BlockSpec((D, D), lambda i: (0, 0)),
          pl.BlockSpec((1, D), lambda i: (0, 0)),
      ],
      out_specs=[_QSPEC, _QSPEC, _QSPEC, _QSPEC],
      out_shape=[jax.ShapeDtypeStruct((N, QW), f32)] * 4,
  )(a01, a23, norm, w, b)


def _layer1_body(a01, a23, norm, w1, b1, w2, g_o):
  h = jnp.concatenate([a01[...], a23[...]], axis=1)
  h = h * norm[...]
  z = jnp.dot(h, w1[...],
              preferred_element_type=f32) + b1[...]
  h2 = jnp.maximum(z, 0.0)
  g = jnp.dot(h2, w2[...],
              preferred_element_type=f32)
  g_o[...] = g * norm[...]


def _layer1(a01, a23, norm, w1, b1, w2):
  return pl.pallas_call(
      _layer1_body,
      grid=(GRID,),
      in_specs=[
          _HSPEC, _HSPEC, _NSPEC,
          pl.BlockSpec((D, D), lambda i: (0, 0)),
          pl.BlockSpec((1, D), lambda i: (0, 0)),
          pl.BlockSpec((D, NCP), lambda i: (0, 0)),
      ],
      out_specs=pl.BlockSpec((BLK, NCP), lambda i: (i, 0)),
      out_shape=jax.ShapeDtypeStruct((N, NCP), f32),
  )(a01, a23, norm, w1, b1, w2)


def _final_body(p, norm, b2, out_o):
  out_o[...] = (p[...][:, :NCP] + p[...][:, NCP:]) * norm[...] + b2[...]


def _final(p, norm, b2):
  return pl.pallas_call(
      _final_body,
      grid=(GRID,),
      in_specs=[
          pl.BlockSpec((BLK, 2 * NCP), lambda i: (i, 0)),
          _NSPEC,
          pl.BlockSpec((1, NCP), lambda i: (0, 0)),
      ],
      out_specs=pl.BlockSpec((BLK, NCP), lambda i: (i, 0)),
      out_shape=jax.ShapeDtypeStruct((N, NCP), f32),
  )(p, norm, b2)


def kernel(x, edge_index, W0, b0, W1, b1, W2, b2):
  src2d = edge_index[0].reshape(NCHUNKS, CHUNK)
  dst2d = edge_index[1].reshape(NCHUNKS, CHUNK)

  W2p = jnp.pad(W2, ((0, 0), (0, NCP - NCLS)))
  b2p = jnp.pad(b2, (0, NCP - NCLS)).reshape(1, NCP)

  zeros16 = jnp.zeros((RPT // 2, DEGW), f32)
  zeros64 = jnp.zeros((RPT // 2, QW), f32)
  ones_hbm = jnp.ones((CHUNK, DEGW), f32)

  deg = _deg_kernel(dst2d, zeros16, ones_hbm)
  norm, h0, h1, h2, h3 = _norm_scale(deg, x)

  a01 = _aggq_kernel(src2d, dst2d, h0, h1, zeros64)
  a23 = _aggq_kernel(src2d, dst2d, h2, h3, zeros64)
  u0, u1, u2, u3 = _layer0(a01, a23, norm, W0, b0.reshape(1, D))

  b01 = _aggq_kernel(src2d, dst2d, u0, u1, zeros64)
  b23 = _aggq_kernel(src2d, dst2d, u2, u3, zeros64)
  g = _layer1(b01, b23, norm, W1, b1.reshape(1, D), W2p)

  p = _agg64_kernel(src2d, dst2d, g, zeros64)
  out = _final(p, norm, b2p)
  return out[:, :NCLS]
